# Initial kernel scaffold; baseline (speedup 1.0000x reference)
import jax
import jax.numpy as jnp
from jax.experimental import pallas as pl


def kernel(ufeats, ifeats, edge_index, edge_type, Wu, Wi, W_h, b_h):
    raise NotImplementedError("write your pallas kernel here")


# speedup vs baseline: 17.2406x; 17.2406x over previous
import functools

import jax
import jax.numpy as jnp
from jax import lax
from jax.experimental import pallas as pl
from jax.experimental.pallas import tpu as pltpu
from jax.experimental.pallas import tpu_sc as plsc

NU = 10000
NI = 10000
NE = 320000
NT = 5
HID = 128
OUTD = 64
NC = 2
NS = 16
L = 16
TAB = NU * NT
TABP = 50176
CH = 80
EPT32 = NE // (NC * NS)
EPT16 = NE // NS
STRIPE = TABP // NS
SUP = 2000

_sds = jax.ShapeDtypeStruct


def _mesh():
    return plsc.VectorSubcoreMesh(
        core_axis_name="c", subcore_axis_name="s",
        num_cores=NC, num_subcores=NS)


@functools.cache
def _deg_kernel():
  return functools.partial(
    pl.kernel,
    out_type=(_sds((NE,), jnp.int32), _sds((NE,), jnp.int32),
              _sds((NC * TABP,), jnp.float32), _sds((NC * TABP,), jnp.float32)),
    mesh=_mesh(),
    scratch_types=[
        pltpu.VMEM((EPT32,), jnp.int32),
        pltpu.VMEM((EPT32,), jnp.int32),
        pltpu.VMEM((EPT32,), jnp.int32),
        pltpu.VMEM((EPT32,), jnp.int32),
        pltpu.VMEM((EPT32,), jnp.int32),
        pltpu.VMEM((CH,), jnp.int32),
        pltpu.VMEM((CH,), jnp.int32),
        pltpu.VMEM((CH,), jnp.float32),
        pltpu.VMEM((STRIPE,), jnp.float32),
        pltpu.VMEM_SHARED((TABP,), jnp.float32),
        pltpu.VMEM_SHARED((TABP,), jnp.float32),
    ],
  )(_deg_body)


def _deg_body(u_hbm, i_hbm, t_hbm, z_hbm, ku_hbm, ki_hbm, degu_hbm, degi_hbm,
              uu, ii, tt, kuf, kif, ku80, ki80, ones80, dbuf,
              degu_s, degi_s):
    cid = lax.axis_index("c")
    sid = lax.axis_index("s")
    wid = cid * NS + sid
    base = wid * EPT32

    one16 = jnp.ones((L,), jnp.float32)
    for g in range(CH // L):
        ones80[pl.ds(g * L, L)] = one16
    pltpu.sync_copy(z_hbm.at[pl.ds(0, STRIPE)], dbuf)
    pltpu.sync_copy(dbuf, degu_s.at[pl.ds(sid * STRIPE, STRIPE)])
    pltpu.sync_copy(dbuf, degi_s.at[pl.ds(sid * STRIPE, STRIPE)])
    plsc.subcore_barrier()

    pltpu.sync_copy(u_hbm.at[pl.ds(base, EPT32)], uu)
    pltpu.sync_copy(i_hbm.at[pl.ds(base, EPT32)], ii)
    pltpu.sync_copy(t_hbm.at[pl.ds(base, EPT32)], tt)

    def chunk(c, _):
        for g in range(CH // L):
            o = c * CH + g * L
            uv = uu[pl.ds(o, L)]
            iv = ii[pl.ds(o, L)]
            tv = tt[pl.ds(o, L)]
            kuv = uv * NT + tv
            kiv = iv * NT + tv
            kuf[pl.ds(o, L)] = kuv
            kif[pl.ds(o, L)] = kiv
            ku80[pl.ds(g * L, L)] = kuv
            ki80[pl.ds(g * L, L)] = kiv
        pltpu.sync_copy(ones80, degu_s.at[ku80], add=True)
        pltpu.sync_copy(ones80, degi_s.at[ki80], add=True)
        return _

    lax.fori_loop(0, EPT32 // CH, chunk, None)
    pltpu.sync_copy(kuf, ku_hbm.at[pl.ds(base, EPT32)])
    pltpu.sync_copy(kif, ki_hbm.at[pl.ds(base, EPT32)])
    plsc.subcore_barrier()

    pltpu.sync_copy(degu_s.at[pl.ds(sid * STRIPE, STRIPE)], dbuf)
    pltpu.sync_copy(dbuf,
                    degu_hbm.at[pl.ds(cid * TABP + sid * STRIPE, STRIPE)])
    pltpu.sync_copy(degi_s.at[pl.ds(sid * STRIPE, STRIPE)], dbuf)
    pltpu.sync_copy(dbuf,
                    degi_hbm.at[pl.ds(cid * TABP + sid * STRIPE, STRIPE)])


def _coef_body(du_ref, di_ref, cu_ref, ci_ref):
    du = du_ref[0] + du_ref[1]
    di = di_ref[0] + di_ref[1]
    cu_ref[...] = lax.rsqrt(jnp.maximum(du, 1.0))
    ci_ref[...] = lax.rsqrt(jnp.maximum(di, 1.0))


def _coef_call(degu, degi):
    r = TABP // HID
    return pl.pallas_call(
        _coef_body,
        out_shape=(_sds((r, HID), jnp.float32), _sds((r, HID), jnp.float32)),
    )(degu.reshape(NC, r, HID), degi.reshape(NC, r, HID))


@functools.cache
def _ce_kernel():
  return functools.partial(
    pl.kernel,
    out_type=_sds((NE,), jnp.float32),
    mesh=_mesh(),
    scratch_types=[
        pltpu.VMEM((EPT32,), jnp.int32),
        pltpu.VMEM((EPT32,), jnp.int32),
        pltpu.VMEM((EPT32,), jnp.float32),
        pltpu.VMEM((CH,), jnp.float32),
        pltpu.VMEM((CH,), jnp.float32),
        pltpu.SemaphoreType.DMA,
    ],
  )(_ce_body)


def _ce_body(ku_hbm, ki_hbm, cu_hbm, ci_hbm, ce_hbm,
             kuf, kif, cef, gu, gi, sem):
    cid = lax.axis_index("c")
    sid = lax.axis_index("s")
    base = (cid * NS + sid) * EPT32
    pltpu.sync_copy(ku_hbm.at[pl.ds(base, EPT32)], kuf)
    pltpu.sync_copy(ki_hbm.at[pl.ds(base, EPT32)], kif)

    def chunk(c, _):
        pltpu.async_copy(cu_hbm.at[kuf.at[pl.ds(c * CH, CH)]], gu, sem).wait()
        pltpu.async_copy(ci_hbm.at[kif.at[pl.ds(c * CH, CH)]], gi, sem).wait()
        for g in range(CH // L):
            cef[pl.ds(c * CH + g * L, L)] = (
                gu[pl.ds(g * L, L)] * gi[pl.ds(g * L, L)])
        return _

    lax.fori_loop(0, EPT32 // CH, chunk, None)
    pltpu.sync_copy(cef, ce_hbm.at[pl.ds(base, EPT32)])


def _bcast_lane(v16, j):
    idx = jnp.full((L, 1), j, jnp.int32)
    dnums = lax.GatherDimensionNumbers(
        offset_dims=(), collapsed_slice_dims=(0,), start_index_map=(0,))
    return lax.gather(v16, idx, dnums, (1,),
                      mode=lax.GatherScatterMode.PROMISE_IN_BOUNDS)


@functools.cache
def _msg_kernel():
  return functools.partial(
    pl.kernel,
    out_type=_sds((NU + NI, HID), jnp.float32),
    mesh=plsc.VectorSubcoreMesh(
        core_axis_name="c", subcore_axis_name="s",
        num_cores=1, num_subcores=NS),
    scratch_types=[
        pltpu.VMEM((SUP,), jnp.int32),
        pltpu.VMEM((SUP,), jnp.int32),
        pltpu.VMEM((SUP,), jnp.float32),
        pltpu.VMEM((CH,), jnp.int32),
        pltpu.VMEM((CH, HID), jnp.float32),
        pltpu.VMEM((CH, HID), jnp.float32),
        pltpu.VMEM_SHARED((NU, HID), jnp.float32),
        pltpu.SemaphoreType.DMA,
    ],
  )(_msg_body)


def _msg_body(hi_hbm, hu_hbm, ki_hbm, ku_hbm, ui_hbm, ii_hbm, ce_hbm, z_hbm,
              out_hbm, gidx, sidx, cef, s80, rows, xbuf, agg_s, sem):
    sid = lax.axis_index("s")
    base = sid * EPT16
    r0 = sid * 640
    nch = jnp.where(sid == NS - 1, 5, 8)

    def run(tab_hbm, g_hbm, s_hbm, out_base):
        pltpu.sync_copy(z_hbm, xbuf)

        def zc(k, _):
            pltpu.sync_copy(xbuf, agg_s.at[pl.ds(r0 + k * CH, CH)])
            return _

        lax.fori_loop(0, nch, zc, None)
        plsc.subcore_barrier()

        def sup(s, _):
            b2 = base + s * SUP
            pltpu.sync_copy(g_hbm.at[pl.ds(b2, SUP)], gidx)
            pltpu.sync_copy(s_hbm.at[pl.ds(b2, SUP)], sidx)
            pltpu.sync_copy(ce_hbm.at[pl.ds(b2, SUP)], cef)

            def chunk(c, _):
                pltpu.async_copy(
                    tab_hbm.at[gidx.at[pl.ds(c * CH, CH)]], rows, sem).wait()
                for g in range(CH // L):
                    s80[pl.ds(g * L, L)] = sidx[pl.ds(c * CH + g * L, L)]
                for g in range(CH // L):
                    cev = cef[pl.ds(c * CH + g * L, L)]
                    for j in range(L):
                        sc = _bcast_lane(cev, j)
                        r = g * L + j
                        for k in range(HID // L):
                            rows[r, pl.ds(k * L, L)] = (
                                rows[r, pl.ds(k * L, L)] * sc)
                pltpu.sync_copy(rows, agg_s.at[s80], add=True)
                return _

            lax.fori_loop(0, SUP // CH, chunk, None)
            return _

        lax.fori_loop(0, EPT16 // SUP, sup, None)
        plsc.subcore_barrier()

        def oc(k, _):
            pltpu.sync_copy(agg_s.at[pl.ds(r0 + k * CH, CH)], xbuf)
            pltpu.sync_copy(
                xbuf, out_hbm.at[pl.ds(out_base + r0 + k * CH, CH)])
            return _

        lax.fori_loop(0, nch, oc, None)
        plsc.subcore_barrier()

    run(hi_hbm, ki_hbm, ui_hbm, 0)
    run(hu_hbm, ku_hbm, ii_hbm, NU)


def _mm_call(x, wstack, bias, relu_flag):
    rtot = x.shape[0]
    d = wstack.shape[-1]
    br = 2000
    nb = rtot // br
    half = nb // 2

    def body(f_ref, x_ref, w_ref, b_ref, o_ref):
        xv = x_ref[...]
        xv = jnp.where(f_ref[0, 0] > 0.5, jnp.maximum(xv, 0.0), xv)
        acc = jnp.dot(xv, w_ref[0], preferred_element_type=jnp.float32,
                      precision=lax.Precision.HIGHEST)
        o_ref[...] = acc + b_ref[...]

    return pl.pallas_call(
        body,
        grid=(nb,),
        in_specs=[
            pl.BlockSpec((1, 1), lambda i: (0, 0)),
            pl.BlockSpec((br, HID), lambda i: (i, 0)),
            pl.BlockSpec((1, HID, d), lambda i: (i // half, 0, 0)),
            pl.BlockSpec((1, d), lambda i: (0, 0)),
        ],
        out_specs=pl.BlockSpec((br, d), lambda i: (i, 0)),
        out_shape=_sds((rtot, d), jnp.float32),
    )(relu_flag, x, wstack, bias)


def kernel(ufeats, ifeats, edge_index, edge_type, Wu, Wi, W_h, b_h):
    u_idx = edge_index[0].astype(jnp.int32)
    i_idx = edge_index[1].astype(jnp.int32)
    et = edge_type.astype(jnp.int32)

    x0 = jnp.concatenate([ufeats, ifeats], axis=0)
    wst = [jnp.stack([Wu[l].transpose(1, 0, 2).reshape(HID, NT * HID),
                      Wi[l].transpose(1, 0, 2).reshape(HID, NT * HID)])
           for l in range(2)]
    zb640 = jnp.zeros((1, NT * HID), jnp.float32)
    wout = jnp.stack([W_h, W_h])
    bout = b_h.reshape(1, OUTD)

    zflat = jnp.zeros((STRIPE,), jnp.float32)
    z2d = jnp.zeros((CH, HID), jnp.float32)

    ku, ki, degu, degi = _deg_kernel()(u_idx, i_idx, et, zflat)
    cu, ci = _coef_call(degu, degi)
    ce = _ce_kernel()(ku, ki, cu.reshape(TABP), ci.reshape(TABP))

    wsc = jnp.stack(wst)
    flags = jnp.array([0.0, 1.0]).reshape(2, 1, 1)

    def layer(x, per):
        w_l, f_l = per
        y = _mm_call(x, w_l, zb640, f_l)
        hu_tab = y[:NU].reshape(TAB, HID)
        hi_tab = y[NU:].reshape(TAB, HID)
        agg = _msg_kernel()(hi_tab, hu_tab, ki, ku, u_idx, i_idx, ce, z2d)
        return agg, None

    aggf, _ = lax.scan(layer, x0, (wsc, flags))
    outc = _mm_call(aggf, wout, bout, jnp.ones((1, 1), jnp.float32))
    return outc[:NU], outc[NU:]


# speedup vs baseline: 40.0365x; 2.3222x over previous
import functools

import jax
import jax.numpy as jnp
from jax import lax
from jax.experimental import pallas as pl
from jax.experimental.pallas import tpu as pltpu
from jax.experimental.pallas import tpu_sc as plsc

NU = 10000
NI = 10000
NE = 320000
NT = 5
HID = 128
OUTD = 64
NC = 2
NS = 16
L = 16
TAB = NU * NT
TABP = 50176
CH = 80
EPT32 = NE // (NC * NS)
EPT16 = NE // NS
STRIPE = TABP // NS
SUP = 4000

_sds = jax.ShapeDtypeStruct


def _mesh():
    return plsc.VectorSubcoreMesh(
        core_axis_name="c", subcore_axis_name="s",
        num_cores=NC, num_subcores=NS)


@functools.cache
def _deg_kernel():
  return functools.partial(
    pl.kernel,
    out_type=(_sds((NE,), jnp.int32), _sds((NE,), jnp.int32),
              _sds((NC * TABP,), jnp.float32), _sds((NC * TABP,), jnp.float32)),
    mesh=_mesh(),
    scratch_types=[
        pltpu.VMEM((EPT32,), jnp.int32),
        pltpu.VMEM((EPT32,), jnp.int32),
        pltpu.VMEM((EPT32,), jnp.int32),
        pltpu.VMEM((EPT32,), jnp.int32),
        pltpu.VMEM((EPT32,), jnp.int32),
        pltpu.VMEM((CH,), jnp.int32),
        pltpu.VMEM((CH,), jnp.int32),
        pltpu.VMEM((CH,), jnp.float32),
        pltpu.VMEM((STRIPE,), jnp.float32),
        pltpu.VMEM_SHARED((TABP,), jnp.float32),
        pltpu.VMEM_SHARED((TABP,), jnp.float32),
    ],
  )(_deg_body)


def _deg_body(u_hbm, i_hbm, t_hbm, z_hbm, ku_hbm, ki_hbm, degu_hbm, degi_hbm,
              uu, ii, tt, kuf, kif, ku80, ki80, ones80, dbuf,
              degu_s, degi_s):
    cid = lax.axis_index("c")
    sid = lax.axis_index("s")
    wid = cid * NS + sid
    base = wid * EPT32

    one16 = jnp.ones((L,), jnp.float32)
    for g in range(CH // L):
        ones80[pl.ds(g * L, L)] = one16
    pltpu.sync_copy(z_hbm.at[pl.ds(0, STRIPE)], dbuf)
    pltpu.sync_copy(dbuf, degu_s.at[pl.ds(sid * STRIPE, STRIPE)])
    pltpu.sync_copy(dbuf, degi_s.at[pl.ds(sid * STRIPE, STRIPE)])
    plsc.subcore_barrier()

    pltpu.sync_copy(u_hbm.at[pl.ds(base, EPT32)], uu)
    pltpu.sync_copy(i_hbm.at[pl.ds(base, EPT32)], ii)
    pltpu.sync_copy(t_hbm.at[pl.ds(base, EPT32)], tt)

    def chunk(c, _):
        for g in range(CH // L):
            o = c * CH + g * L
            uv = uu[pl.ds(o, L)]
            iv = ii[pl.ds(o, L)]
            tv = tt[pl.ds(o, L)]
            kuv = uv * NT + tv
            kiv = iv * NT + tv
            kuf[pl.ds(o, L)] = kuv
            kif[pl.ds(o, L)] = kiv
            ku80[pl.ds(g * L, L)] = kuv
            ki80[pl.ds(g * L, L)] = kiv
        pltpu.sync_copy(ones80, degu_s.at[ku80], add=True)
        pltpu.sync_copy(ones80, degi_s.at[ki80], add=True)
        return _

    lax.fori_loop(0, EPT32 // CH, chunk, None)
    pltpu.sync_copy(kuf, ku_hbm.at[pl.ds(base, EPT32)])
    pltpu.sync_copy(kif, ki_hbm.at[pl.ds(base, EPT32)])
    plsc.subcore_barrier()

    pltpu.sync_copy(degu_s.at[pl.ds(sid * STRIPE, STRIPE)], dbuf)
    pltpu.sync_copy(dbuf,
                    degu_hbm.at[pl.ds(cid * TABP + sid * STRIPE, STRIPE)])
    pltpu.sync_copy(degi_s.at[pl.ds(sid * STRIPE, STRIPE)], dbuf)
    pltpu.sync_copy(dbuf,
                    degi_hbm.at[pl.ds(cid * TABP + sid * STRIPE, STRIPE)])


def _coef_body(du_ref, di_ref, cu_ref, ci_ref):
    du = du_ref[0] + du_ref[1]
    di = di_ref[0] + di_ref[1]
    cu_ref[...] = lax.rsqrt(jnp.maximum(du, 1.0))
    ci_ref[...] = lax.rsqrt(jnp.maximum(di, 1.0))


def _coef_call(degu, degi):
    r = TABP // HID
    return pl.pallas_call(
        _coef_body,
        out_shape=(_sds((r, HID), jnp.float32), _sds((r, HID), jnp.float32)),
    )(degu.reshape(NC, r, HID), degi.reshape(NC, r, HID))


@functools.cache
def _ce_kernel():
  return functools.partial(
    pl.kernel,
    out_type=_sds((NE,), jnp.float32),
    mesh=_mesh(),
    scratch_types=[
        pltpu.VMEM((EPT32,), jnp.int32),
        pltpu.VMEM((EPT32,), jnp.int32),
        pltpu.VMEM((EPT32,), jnp.float32),
        pltpu.VMEM((CH,), jnp.float32),
        pltpu.VMEM((CH,), jnp.float32),
        pltpu.SemaphoreType.DMA,
    ],
  )(_ce_body)


def _ce_body(ku_hbm, ki_hbm, cu_hbm, ci_hbm, ce_hbm,
             kuf, kif, cef, gu, gi, sem):
    cid = lax.axis_index("c")
    sid = lax.axis_index("s")
    base = (cid * NS + sid) * EPT32
    pltpu.sync_copy(ku_hbm.at[pl.ds(base, EPT32)], kuf)
    pltpu.sync_copy(ki_hbm.at[pl.ds(base, EPT32)], kif)

    def chunk(c, _):
        pltpu.async_copy(cu_hbm.at[kuf.at[pl.ds(c * CH, CH)]], gu, sem).wait()
        pltpu.async_copy(ci_hbm.at[kif.at[pl.ds(c * CH, CH)]], gi, sem).wait()
        for g in range(CH // L):
            cef[pl.ds(c * CH + g * L, L)] = (
                gu[pl.ds(g * L, L)] * gi[pl.ds(g * L, L)])
        return _

    lax.fori_loop(0, EPT32 // CH, chunk, None)
    pltpu.sync_copy(cef, ce_hbm.at[pl.ds(base, EPT32)])


def _bcast_lane(v16, j):
    idx = jnp.full((L, 1), j, jnp.int32)
    dnums = lax.GatherDimensionNumbers(
        offset_dims=(), collapsed_slice_dims=(0,), start_index_map=(0,))
    return lax.gather(v16, idx, dnums, (1,),
                      mode=lax.GatherScatterMode.PROMISE_IN_BOUNDS)


@functools.cache
def _msg_kernel():
  return functools.partial(
    pl.kernel,
    out_type=_sds((NU + NI, HID), jnp.float32),
    mesh=_mesh(),
    scratch_types=[
        pltpu.VMEM((SUP,), jnp.int32),
        pltpu.VMEM((SUP,), jnp.int32),
        pltpu.VMEM((SUP,), jnp.float32),
        pltpu.VMEM((CH,), jnp.int32),
        pltpu.VMEM((CH, HID), jnp.float32),
        pltpu.VMEM((CH, HID), jnp.float32),
        pltpu.VMEM((CH, HID), jnp.float32),
        pltpu.VMEM_SHARED((NU, HID), jnp.float32),
        pltpu.SemaphoreType.DMA,
        pltpu.SemaphoreType.DMA,
    ],
  )(_msg_body)


def _msg_body(hi_hbm, hu_hbm, ki_hbm, ku_hbm, ui_hbm, ii_hbm, ce_hbm, z_hbm,
              out_hbm, gidx, sidx, cef, s80, rows_a, rows_b, xbuf, agg_s,
              sem_a, sem_b):
    cid = lax.axis_index("c")
    sid = lax.axis_index("s")
    base = sid * EPT16
    r0 = sid * 640
    nch = jnp.where(sid == NS - 1, 5, 8)
    npair = SUP // (2 * CH)

    def run(tab_hbm, g_hbm, s_hbm, out_base):
        pltpu.sync_copy(z_hbm, xbuf)

        def zc(k, _):
            pltpu.sync_copy(xbuf, agg_s.at[pl.ds(r0 + k * CH, CH)])
            return _

        lax.fori_loop(0, nch, zc, None)
        plsc.subcore_barrier()

        def scale_and_scatter(buf, cbase):
            def grp(g, _):
                cev = cef[pl.ds(cbase + g * L, L)]
                s80[pl.ds(g * L, L)] = sidx[pl.ds(cbase + g * L, L)]
                for j in range(L):
                    sc = _bcast_lane(cev, j)
                    r = g * L + j
                    for k in range(HID // L):
                        buf[r, pl.ds(k * L, L)] = buf[r, pl.ds(k * L, L)] * sc
                return _

            lax.fori_loop(0, CH // L, grp, None)
            pltpu.sync_copy(buf, agg_s.at[s80], add=True)

        def sup(s, _):
            b2 = base + s * SUP
            pltpu.sync_copy(g_hbm.at[pl.ds(b2, SUP)], gidx)
            pltpu.sync_copy(s_hbm.at[pl.ds(b2, SUP)], sidx)
            pltpu.sync_copy(ce_hbm.at[pl.ds(b2, SUP)], cef)
            pltpu.async_copy(
                tab_hbm.at[gidx.at[pl.ds(0, CH)]], rows_a, sem_a)

            def pair(p, _):
                c0 = 2 * p * CH
                c1 = c0 + CH
                pltpu.make_async_copy(
                    tab_hbm.at[gidx.at[pl.ds(c0, CH)]], rows_a, sem_a).wait()
                pltpu.async_copy(
                    tab_hbm.at[gidx.at[pl.ds(c1, CH)]], rows_b, sem_b)
                scale_and_scatter(rows_a, c0)

                @pl.when(p < npair - 1)
                def _():
                    pltpu.async_copy(
                        tab_hbm.at[gidx.at[pl.ds(c1 + CH, CH)]], rows_a,
                        sem_a)

                pltpu.make_async_copy(
                    tab_hbm.at[gidx.at[pl.ds(c1, CH)]], rows_b, sem_b).wait()
                scale_and_scatter(rows_b, c1)
                return _

            lax.fori_loop(0, npair, pair, None)
            return _

        lax.fori_loop(0, EPT16 // SUP, sup, None)
        plsc.subcore_barrier()

        def oc(k, _):
            pltpu.sync_copy(agg_s.at[pl.ds(r0 + k * CH, CH)], xbuf)
            pltpu.sync_copy(
                xbuf, out_hbm.at[pl.ds(out_base + r0 + k * CH, CH)])
            return _

        lax.fori_loop(0, nch, oc, None)
        plsc.subcore_barrier()

    @pl.when(cid == 0)
    def _():
        run(hi_hbm, ki_hbm, ui_hbm, 0)

    @pl.when(cid == 1)
    def _():
        run(hu_hbm, ku_hbm, ii_hbm, NU)


def _mm_call(x, wstack, bias, relu_flag):
    rtot = x.shape[0]
    d = wstack.shape[-1]
    br = 2000
    nb = rtot // br
    half = nb // 2

    def body(f_ref, x_ref, w_ref, b_ref, o_ref):
        xv = x_ref[...]
        xv = jnp.where(f_ref[0, 0] > 0.5, jnp.maximum(xv, 0.0), xv)
        acc = jnp.dot(xv, w_ref[0], preferred_element_type=jnp.float32,
                      precision=lax.Precision.HIGHEST)
        o_ref[...] = acc + b_ref[...]

    return pl.pallas_call(
        body,
        grid=(nb,),
        in_specs=[
            pl.BlockSpec((1, 1), lambda i: (0, 0)),
            pl.BlockSpec((br, HID), lambda i: (i, 0)),
            pl.BlockSpec((1, HID, d), lambda i: (i // half, 0, 0)),
            pl.BlockSpec((1, d), lambda i: (0, 0)),
        ],
        out_specs=pl.BlockSpec((br, d), lambda i: (i, 0)),
        out_shape=_sds((rtot, d), jnp.float32),
    )(relu_flag, x, wstack, bias)


def kernel(ufeats, ifeats, edge_index, edge_type, Wu, Wi, W_h, b_h):
    u_idx = edge_index[0].astype(jnp.int32)
    i_idx = edge_index[1].astype(jnp.int32)
    et = edge_type.astype(jnp.int32)

    x0 = jnp.concatenate([ufeats, ifeats], axis=0)
    wst = [jnp.stack([Wu[l].transpose(1, 0, 2).reshape(HID, NT * HID),
                      Wi[l].transpose(1, 0, 2).reshape(HID, NT * HID)])
           for l in range(2)]
    zb640 = jnp.zeros((1, NT * HID), jnp.float32)
    wout = jnp.stack([W_h, W_h])
    bout = b_h.reshape(1, OUTD)

    zflat = jnp.zeros((STRIPE,), jnp.float32)
    z2d = jnp.zeros((CH, HID), jnp.float32)

    ku, ki, degu, degi = _deg_kernel()(u_idx, i_idx, et, zflat)
    cu, ci = _coef_call(degu, degi)
    ce = _ce_kernel()(ku, ki, cu.reshape(TABP), ci.reshape(TABP))

    wsc = jnp.stack(wst)
    flags = jnp.array([0.0, 1.0]).reshape(2, 1, 1)

    def layer(x, per):
        w_l, f_l = per
        y = _mm_call(x, w_l, zb640, f_l)
        hu_tab = y[:NU].reshape(TAB, HID)
        hi_tab = y[NU:].reshape(TAB, HID)
        agg = _msg_kernel()(hi_tab, hu_tab, ki, ku, u_idx, i_idx, ce, z2d)
        return agg, None

    aggf, _ = lax.scan(layer, x0, (wsc, flags))
    outc = _mm_call(aggf, wout, bout, jnp.ones((1, 1), jnp.float32))
    return outc[:NU], outc[NU:]
